# baseline (speedup 1.0000x reference)
import functools

import jax
import jax.numpy as jnp
from jax import lax
from jax.experimental import pallas as pl
from jax.experimental.pallas import tpu as pltpu
from jax.experimental.pallas import tpu_sc as plsc

NC = 2
NS = 16
LANES = 16
NW = NC * NS


@functools.lru_cache(maxsize=None)
def _make_gs_kernel(N, C, D, H, W, SQ):
    DHW = D * H * W
    HW = H * W
    Q = N * SQ
    assert Q % NW == 0
    TQ = Q // NW
    BQ = 128
    NG = BQ // LANES
    NB = TQ // BQ
    assert SQ % TQ == 0, "tile span must stay within one batch"
    assert TQ % BQ == 0 and NB % 2 == 0 and NB >= 4
    TILES_PER_N = SQ // TQ
    NBTOT = Q // BQ

    mesh = plsc.VectorSubcoreMesh(core_axis_name="c", subcore_axis_name="s",
                                  num_cores=NC, num_subcores=NS)

    @functools.partial(
        pl.kernel,
        out_type=jax.ShapeDtypeStruct((N * C, SQ), jnp.float32),
        mesh=mesh,
        scratch_types=[
            pltpu.VMEM((3 * BQ,), jnp.float32),
            pltpu.VMEM((3 * BQ,), jnp.float32),
            pltpu.VMEM((NG, 8 * LANES), jnp.int32),
            pltpu.VMEM((NG, 8 * LANES), jnp.int32),
            pltpu.VMEM((NG, 8 * LANES), jnp.float32),
            pltpu.VMEM((NG, 8 * LANES), jnp.float32),
            pltpu.VMEM((BQ * 8, C), jnp.float32),
            pltpu.VMEM((BQ * 8, C), jnp.float32),
            pltpu.VMEM((C, BQ), jnp.float32),
            pltpu.VMEM((C, BQ), jnp.float32),
            pltpu.SemaphoreType.DMA,
            pltpu.SemaphoreType.DMA,
            pltpu.SemaphoreType.DMA,
            pltpu.SemaphoreType.DMA,
            pltpu.SemaphoreType.DMA,
            pltpu.SemaphoreType.DMA,
        ],
        compiler_params=pltpu.CompilerParams(needs_layout_passes=False,
                                             use_tc_tiling_on_sc=False),
    )
    def gs_kernel(tab_h, coords_h, out_h,
                  cb0, cb1, idx0, idx1, w0, w1, rows0, rows1, ob0, ob1,
                  csem0, csem1, gsem0, gsem1, osem0, osem1):
        cbuf = (cb0, cb1)
        idxb = (idx0, idx1)
        wbuf = (w0, w1)
        rowsb = (rows0, rows1)
        outbb = (ob0, ob1)
        csem = (csem0, csem1)
        gsem = (gsem0, gsem1)
        osem = (osem0, osem1)

        cid = lax.axis_index("c")
        sid = lax.axis_index("s")
        wid = cid * NS + sid
        n = wid // TILES_PER_N
        sq_base = (wid % TILES_PER_N) * TQ
        blk_base = wid * NB
        iota = lax.iota(jnp.int32, LANES)
        base = n * DHW

        def floor_split(v):
            vi = v.astype(jnp.int32)
            vf = vi.astype(jnp.float32)
            v0 = jnp.where(vf > v, vi - 1, vi)
            f = v - v0.astype(jnp.float32)
            return v0, f

        def axis_terms(c0, f, L, stride):
            c1 = c0 + 1
            wlo = jnp.where((c0 >= 0) & (c0 <= L - 1), 1.0 - f, 0.0)
            whi = jnp.where((c1 >= 0) & (c1 <= L - 1), f, 0.0)
            ilo = jnp.clip(c0, 0, L - 1) * stride
            ihi = jnp.clip(c1, 0, L - 1) * stride
            return (wlo, whi), (ilo, ihi)

        def compute_idx(cb, idxr, wr):
            def idx_body(g, carry):
                qv3 = (g * LANES + iota) * 3
                x = (plsc.load_gather(cb, [qv3]) + 1.0) * (W * 0.5) - 0.5
                y = (plsc.load_gather(cb, [qv3 + 1]) + 1.0) * (H * 0.5) - 0.5
                z = (plsc.load_gather(cb, [qv3 + 2]) + 1.0) * (D * 0.5) - 0.5
                x0, fx = floor_split(x)
                y0, fy = floor_split(y)
                z0, fz = floor_split(z)
                wx, ix = axis_terms(x0, fx, W, 1)
                wy, iy = axis_terms(y0, fy, H, W)
                wz, iz = axis_terms(z0, fz, D, HW)
                for c in range(8):
                    zs, ys, xs = (c >> 2) & 1, (c >> 1) & 1, c & 1
                    idxr[g, pl.ds(c * LANES, LANES)] = (
                        base + iz[zs] + iy[ys] + ix[xs])
                    wr[g, pl.ds(c * LANES, LANES)] = wz[zs] * wy[ys] * wx[xs]
                return carry
            lax.fori_loop(0, NG, idx_body, 0, unroll=False)

        def issue_gathers(idxr, rowsr, sem):
            for g in range(NG):
                pltpu.async_copy(
                    tab_h.at[idxr.at[g]],
                    rowsr.at[pl.ds(g * 8 * LANES, 8 * LANES)],
                    sem)

        def drain_gathers(rowsr, sem):
            pltpu.make_async_copy(tab_h.at[pl.ds(0, BQ * 8)], rowsr,
                                  sem).wait()

        doffs = [(d + iota) & (LANES - 1) for d in range(LANES)]

        def compute_sum(wr, rowsr, outr):
            def sum_body(g, carry):
                qvec = g * LANES + iota
                wrow = [wr[g, pl.ds(c * LANES, LANES)] for c in range(8)]
                rterm = [g * (8 * LANES) + c * LANES + iota for c in range(8)]
                for h in range(C // LANES):
                    for d in range(LANES):
                        choff = doffs[d] + h * LANES
                        acc = wrow[0] * plsc.load_gather(
                            rowsr, [rterm[0], choff])
                        for c in range(1, 8):
                            acc = acc + wrow[c] * plsc.load_gather(
                                rowsr, [rterm[c], choff])
                        plsc.store_scatter(outr, [choff, qvec], acc)
                return carry
            lax.fori_loop(0, NG, sum_body, 0, unroll=False)

        def out_slice(b):
            q0 = sq_base + b * BQ
            return out_h.at[pl.ds(n * C, C), pl.ds(q0, BQ)]

        def coords_slice(b):
            gq0 = (blk_base + b) * BQ
            return coords_h.at[pl.ds(gq0 * 3, 3 * BQ)]

        pltpu.sync_copy(coords_slice(0), cb0)
        compute_idx(cb0, idx0, w0)
        issue_gathers(idx0, rows0, gsem0)
        pltpu.async_copy(coords_slice(1), cb1, csem1)

        def outer(ob, carry):
            for par in (0, 1):
                b = ob * 2 + par
                X, Y = par, 1 - par

                @pl.when(b + 1 < NB)
                def _():
                    pltpu.make_async_copy(coords_slice(0),
                                          cbuf[Y], csem[Y]).wait()
                    compute_idx(cbuf[Y], idxb[Y], wbuf[Y])
                    issue_gathers(idxb[Y], rowsb[Y], gsem[Y])

                @pl.when(b + 2 < NB)
                def _():
                    pltpu.async_copy(coords_slice(b + 2), cbuf[X], csem[X])

                drain_gathers(rowsb[X], gsem[X])

                @pl.when(b >= 2)
                def _():
                    pltpu.make_async_copy(outbb[X], out_slice(0),
                                          osem[X]).wait()

                compute_sum(wbuf[X], rowsb[X], outbb[X])
                pltpu.async_copy(outbb[X], out_slice(b), osem[X])
            return carry

        lax.fori_loop(0, NB // 2, outer, 0, unroll=False)

        pltpu.make_async_copy(outbb[0], out_slice(0), osem0).wait()
        pltpu.make_async_copy(outbb[1], out_slice(0), osem1).wait()

    return gs_kernel


def kernel(img, grid):
    N, C, D, H, W = img.shape
    N2, Do, Ho, Wo, three = grid.shape
    assert N2 == N and three == 3
    SQ = Do * Ho * Wo
    Q = N * SQ
    BQ = 128
    tab = jnp.moveaxis(img, 1, -1).reshape(N * D * H * W, C)
    f = _make_gs_kernel(N, C, D, H, W, SQ)
    out = f(tab, grid.reshape(Q * 3))
    return out.reshape(N, C, Do, Ho, Wo)


# speedup vs baseline: 1.7730x; 1.7730x over previous
import functools

import jax
import jax.numpy as jnp
from jax import lax
from jax.experimental import pallas as pl
from jax.experimental.pallas import tpu as pltpu
from jax.experimental.pallas import tpu_sc as plsc

NC = 2
NS = 16
LANES = 16
NW = NC * NS


@functools.lru_cache(maxsize=None)
def _make_gs_kernel(N, C, D, H, W, SQ):
    DHW = D * H * W
    HW = H * W
    Q = N * SQ
    assert Q % NW == 0
    TQ = Q // NW
    BQ = 128
    NG = BQ // LANES
    NB = TQ // BQ
    assert SQ % TQ == 0, "tile span must stay within one batch"
    assert TQ % BQ == 0 and NB % 2 == 0 and NB >= 4
    TILES_PER_N = SQ // TQ
    NBTOT = Q // BQ

    mesh = plsc.VectorSubcoreMesh(core_axis_name="c", subcore_axis_name="s",
                                  num_cores=NC, num_subcores=NS)

    @functools.partial(
        pl.kernel,
        out_type=jax.ShapeDtypeStruct((N * C, SQ), jnp.float32),
        mesh=mesh,
        scratch_types=[
            pltpu.VMEM((3 * BQ,), jnp.float32),
            pltpu.VMEM((3 * BQ,), jnp.float32),
            pltpu.VMEM((NG, 8 * LANES), jnp.int32),
            pltpu.VMEM((NG, 8 * LANES), jnp.int32),
            pltpu.VMEM((NG, 8 * LANES), jnp.float32),
            pltpu.VMEM((NG, 8 * LANES), jnp.float32),
            pltpu.VMEM((BQ * 8, C), jnp.float32),
            pltpu.VMEM((BQ * 8, C), jnp.float32),
            pltpu.VMEM((C, BQ), jnp.float32),
            pltpu.VMEM((C, BQ), jnp.float32),
            pltpu.SemaphoreType.DMA,
            pltpu.SemaphoreType.DMA,
            pltpu.SemaphoreType.DMA,
            pltpu.SemaphoreType.DMA,
            pltpu.SemaphoreType.DMA,
            pltpu.SemaphoreType.DMA,
        ],
        compiler_params=pltpu.CompilerParams(needs_layout_passes=False,
                                             use_tc_tiling_on_sc=False),
    )
    def gs_kernel(tab_h, coords_h, out_h,
                  cb0, cb1, idx0, idx1, w0, w1, rows0, rows1, ob0, ob1,
                  csem0, csem1, gsem0, gsem1, osem0, osem1):
        cbuf = (cb0, cb1)
        idxb = (idx0, idx1)
        wbuf = (w0, w1)
        rowsb = (rows0, rows1)
        outbb = (ob0, ob1)
        csem = (csem0, csem1)
        gsem = (gsem0, gsem1)
        osem = (osem0, osem1)

        cid = lax.axis_index("c")
        sid = lax.axis_index("s")
        wid = cid * NS + sid
        n = wid // TILES_PER_N
        sq_base = (wid % TILES_PER_N) * TQ
        blk_base = wid * NB
        iota = lax.iota(jnp.int32, LANES)
        base = n * DHW

        def floor_split(v):
            vi = v.astype(jnp.int32)
            vf = vi.astype(jnp.float32)
            v0 = jnp.where(vf > v, vi - 1, vi)
            f = v - v0.astype(jnp.float32)
            return v0, f

        def axis_terms(c0, f, L, stride):
            c1 = c0 + 1
            wlo = jnp.where((c0 >= 0) & (c0 <= L - 1), 1.0 - f, 0.0)
            whi = jnp.where((c1 >= 0) & (c1 <= L - 1), f, 0.0)
            ilo = jnp.clip(c0, 0, L - 1) * stride
            ihi = jnp.clip(c1, 0, L - 1) * stride
            return (wlo, whi), (ilo, ihi)

        def compute_idx(cb, idxr, wr):
            def idx_body(g, carry):
                sl = pl.ds(g * LANES, LANES)
                x = (cb[sl] + 1.0) * (W * 0.5) - 0.5
                y = (cb[pl.ds(BQ + g * LANES, LANES)] + 1.0) * (H * 0.5) - 0.5
                z = (cb[pl.ds(2 * BQ + g * LANES, LANES)] + 1.0) * (D * 0.5) - 0.5
                x0, fx = floor_split(x)
                y0, fy = floor_split(y)
                z0, fz = floor_split(z)
                wx, ix = axis_terms(x0, fx, W, 1)
                wy, iy = axis_terms(y0, fy, H, W)
                wz, iz = axis_terms(z0, fz, D, HW)
                for c in range(8):
                    zs, ys, xs = (c >> 2) & 1, (c >> 1) & 1, c & 1
                    idxr[g, pl.ds(c * LANES, LANES)] = (
                        base + iz[zs] + iy[ys] + ix[xs])
                    wr[g, pl.ds(c * LANES, LANES)] = wz[zs] * wy[ys] * wx[xs]
                return carry
            lax.fori_loop(0, NG, idx_body, 0, unroll=False)

        def issue_gathers(idxr, rowsr, sem):
            for g in range(NG):
                pltpu.async_copy(
                    tab_h.at[idxr.at[g]],
                    rowsr.at[pl.ds(g * 8 * LANES, 8 * LANES)],
                    sem)

        def drain_gathers(rowsr, sem):
            pltpu.make_async_copy(tab_h.at[pl.ds(0, BQ * 8)], rowsr,
                                  sem).wait()

        doffs = [(d + iota) & (LANES - 1) for d in range(LANES)]

        def compute_sum(wr, rowsr, outr):
            def sum_body(g, carry):
                qvec = g * LANES + iota
                wrow = [wr[g, pl.ds(c * LANES, LANES)] for c in range(8)]
                rterm = [g * (8 * LANES) + c * LANES + iota for c in range(8)]
                for h in range(C // LANES):
                    for d in range(LANES):
                        choff = doffs[d] + h * LANES
                        acc = wrow[0] * plsc.load_gather(
                            rowsr, [rterm[0], choff])
                        for c in range(1, 8):
                            acc = acc + wrow[c] * plsc.load_gather(
                                rowsr, [rterm[c], choff])
                        plsc.store_scatter(outr, [choff, qvec], acc)
                return carry
            lax.fori_loop(0, NG, sum_body, 0, unroll=False)

        def out_slice(b):
            q0 = sq_base + b * BQ
            return out_h.at[pl.ds(n * C, C), pl.ds(q0, BQ)]

        def coords_slice(b):
            return coords_h.at[blk_base + b]

        pltpu.sync_copy(coords_slice(0), cb0)
        compute_idx(cb0, idx0, w0)
        issue_gathers(idx0, rows0, gsem0)
        pltpu.async_copy(coords_slice(1), cb1, csem1)

        def outer(ob, carry):
            for par in (0, 1):
                b = ob * 2 + par
                X, Y = par, 1 - par

                @pl.when(b + 1 < NB)
                def _():
                    pltpu.make_async_copy(coords_slice(0),
                                          cbuf[Y], csem[Y]).wait()
                    compute_idx(cbuf[Y], idxb[Y], wbuf[Y])
                    issue_gathers(idxb[Y], rowsb[Y], gsem[Y])

                @pl.when(b + 2 < NB)
                def _():
                    pltpu.async_copy(coords_slice(b + 2), cbuf[X], csem[X])

                drain_gathers(rowsb[X], gsem[X])

                @pl.when(b >= 2)
                def _():
                    pltpu.make_async_copy(outbb[X], out_slice(0),
                                          osem[X]).wait()

                compute_sum(wbuf[X], rowsb[X], outbb[X])
                pltpu.async_copy(outbb[X], out_slice(b), osem[X])
            return carry

        lax.fori_loop(0, NB // 2, outer, 0, unroll=False)

        pltpu.make_async_copy(outbb[0], out_slice(0), osem0).wait()
        pltpu.make_async_copy(outbb[1], out_slice(0), osem1).wait()

    return gs_kernel


@functools.lru_cache(maxsize=None)
def _make_transpose_kernel(N, C, D, H, W):
    HW = H * W
    DHW = D * HW
    PK = 128 // C
    RB = HW // PK

    def body(in_ref, out_ref, scr):
        x = in_ref[0, :, 0]
        x2 = x.reshape(C, HW)
        scr[...] = x2.T
        for xoff in range(PK):
            out_ref[:, xoff * C:(xoff + 1) * C] = scr[
                pl.Slice(xoff, RB, PK), :]

    return pl.pallas_call(
        body,
        grid=(N, D),
        in_specs=[pl.BlockSpec((1, C, 1, H, W), lambda n, d: (n, 0, d, 0, 0))],
        out_specs=pl.BlockSpec((RB, PK * C), lambda n, d: (n * D + d, 0)),
        out_shape=jax.ShapeDtypeStruct((N * DHW // PK, PK * C), jnp.float32),
        scratch_shapes=[pltpu.VMEM((HW, C), jnp.float32)],
    )


def kernel(img, grid):
    N, C, D, H, W = img.shape
    N2, Do, Ho, Wo, three = grid.shape
    assert N2 == N and three == 3
    SQ = Do * Ho * Wo
    Q = N * SQ
    BQ = 128
    DHW = D * H * W
    assert (DHW * C) % 128 == 0 and 128 % C == 0
    tab = _make_transpose_kernel(N, C, D, H, W)(img).reshape(N * DHW, C)
    g3 = jnp.moveaxis(grid, -1, 0).reshape(3, Q)
    coords = jnp.moveaxis(g3.reshape(3, Q // BQ, BQ), 0, 1).reshape(
        Q // BQ, 3 * BQ)
    f = _make_gs_kernel(N, C, D, H, W, SQ)
    out = f(tab, coords)
    return out.reshape(N, C, Do, Ho, Wo)


# speedup vs baseline: 1.7752x; 1.0012x over previous
import functools

import jax
import jax.numpy as jnp
from jax import lax
from jax.experimental import pallas as pl
from jax.experimental.pallas import tpu as pltpu
from jax.experimental.pallas import tpu_sc as plsc

NC = 2
NS = 16
LANES = 16
NW = NC * NS


@functools.lru_cache(maxsize=None)
def _make_gs_kernel(N, C, D, H, W, SQ):
    DHW = D * H * W
    HW = H * W
    Q = N * SQ
    assert Q % NW == 0
    TQ = Q // NW
    BQ = 128
    NG = BQ // LANES
    NB = TQ // BQ
    assert SQ % TQ == 0, "tile span must stay within one batch"
    assert TQ % BQ == 0 and NB % 2 == 0 and NB >= 4
    PK = 128 // C
    assert C == 32 and W == 64 and HW % 512 == 0
    TILES_PER_N = SQ // TQ
    NBTOT = Q // BQ

    mesh = plsc.VectorSubcoreMesh(core_axis_name="c", subcore_axis_name="s",
                                  num_cores=NC, num_subcores=NS)

    @functools.partial(
        pl.kernel,
        out_type=jax.ShapeDtypeStruct((N * C, SQ), jnp.float32),
        mesh=mesh,
        scratch_types=[
            pltpu.VMEM((3 * BQ,), jnp.float32),
            pltpu.VMEM((3 * BQ,), jnp.float32),
            pltpu.VMEM((NG, 8 * LANES), jnp.int32),
            pltpu.VMEM((NG, 8 * LANES), jnp.int32),
            pltpu.VMEM((NG, 8 * LANES), jnp.float32),
            pltpu.VMEM((NG, 8 * LANES), jnp.float32),
            pltpu.VMEM((BQ * 8, C), jnp.float32),
            pltpu.VMEM((BQ * 8, C), jnp.float32),
            pltpu.VMEM((C, BQ), jnp.float32),
            pltpu.VMEM((C, BQ), jnp.float32),
            pltpu.SemaphoreType.DMA,
            pltpu.SemaphoreType.DMA,
            pltpu.SemaphoreType.DMA,
            pltpu.SemaphoreType.DMA,
            pltpu.SemaphoreType.DMA,
            pltpu.SemaphoreType.DMA,
        ],
        compiler_params=pltpu.CompilerParams(needs_layout_passes=False,
                                             use_tc_tiling_on_sc=False),
    )
    def gs_kernel(tab_h, coords_h, out_h,
                  cb0, cb1, idx0, idx1, w0, w1, rows0, rows1, ob0, ob1,
                  csem0, csem1, gsem0, gsem1, osem0, osem1):
        cbuf = (cb0, cb1)
        idxb = (idx0, idx1)
        wbuf = (w0, w1)
        rowsb = (rows0, rows1)
        outbb = (ob0, ob1)
        csem = (csem0, csem1)
        gsem = (gsem0, gsem1)
        osem = (osem0, osem1)

        cid = lax.axis_index("c")
        sid = lax.axis_index("s")
        wid = cid * NS + sid
        n = wid // TILES_PER_N
        sq_base = (wid % TILES_PER_N) * TQ
        blk_base = wid * NB
        iota = lax.iota(jnp.int32, LANES)
        base = n * DHW

        def floor_split(v):
            vi = v.astype(jnp.int32)
            vf = vi.astype(jnp.float32)
            v0 = jnp.where(vf > v, vi - 1, vi)
            f = v - v0.astype(jnp.float32)
            return v0, f

        def axis_terms(c0, f, L, stride):
            c1 = c0 + 1
            wlo = jnp.where((c0 >= 0) & (c0 <= L - 1), 1.0 - f, 0.0)
            whi = jnp.where((c1 >= 0) & (c1 <= L - 1), f, 0.0)
            ilo = jnp.clip(c0, 0, L - 1) * stride
            ihi = jnp.clip(c1, 0, L - 1) * stride
            return (wlo, whi), (ilo, ihi)

        def compute_idx(cb, idxr, wr):
            def idx_body(g, carry):
                sl = pl.ds(g * LANES, LANES)
                x = (cb[sl] + 1.0) * (W * 0.5) - 0.5
                y = (cb[pl.ds(BQ + g * LANES, LANES)] + 1.0) * (H * 0.5) - 0.5
                z = (cb[pl.ds(2 * BQ + g * LANES, LANES)] + 1.0) * (D * 0.5) - 0.5
                x0, fx = floor_split(x)
                y0, fy = floor_split(y)
                z0, fz = floor_split(z)
                wx, ix = axis_terms(x0, fx, W, PK)
                wy, iyc = axis_terms(y0, fy, H, 1)
                wz, iz = axis_terms(z0, fz, D, HW)
                iy = [(yc >> 3) * (2 * W * PK) + (yc & 1) * (PK * W)
                      + ((yc >> 1) & 3) for yc in iyc]
                for c in range(8):
                    zs, ys, xs = (c >> 2) & 1, (c >> 1) & 1, c & 1
                    idxr[g, pl.ds(c * LANES, LANES)] = (
                        base + iz[zs] + iy[ys] + ix[xs])
                    wr[g, pl.ds(c * LANES, LANES)] = wz[zs] * wy[ys] * wx[xs]
                return carry
            lax.fori_loop(0, NG, idx_body, 0, unroll=False)

        def issue_gathers(idxr, rowsr, sem):
            for g in range(NG):
                pltpu.async_copy(
                    tab_h.at[idxr.at[g]],
                    rowsr.at[pl.ds(g * 8 * LANES, 8 * LANES)],
                    sem)

        def drain_gathers(rowsr, sem):
            pltpu.make_async_copy(tab_h.at[pl.ds(0, BQ * 8)], rowsr,
                                  sem).wait()

        doffs = [(d + iota) & (LANES - 1) for d in range(LANES)]

        def compute_sum(wr, rowsr, outr):
            def sum_body(g, carry):
                qvec = g * LANES + iota
                wrow = [wr[g, pl.ds(c * LANES, LANES)] for c in range(8)]
                rterm = [g * (8 * LANES) + c * LANES + iota for c in range(8)]
                for h in range(C // LANES):
                    for d in range(LANES):
                        choff = doffs[d] + h * LANES
                        acc = wrow[0] * plsc.load_gather(
                            rowsr, [rterm[0], choff])
                        for c in range(1, 8):
                            acc = acc + wrow[c] * plsc.load_gather(
                                rowsr, [rterm[c], choff])
                        plsc.store_scatter(outr, [choff, qvec], acc)
                return carry
            lax.fori_loop(0, NG, sum_body, 0, unroll=False)

        def out_slice(b):
            q0 = sq_base + b * BQ
            return out_h.at[pl.ds(n * C, C), pl.ds(q0, BQ)]

        def coords_slice(b):
            return coords_h.at[blk_base + b]

        pltpu.sync_copy(coords_slice(0), cb0)
        compute_idx(cb0, idx0, w0)
        issue_gathers(idx0, rows0, gsem0)
        pltpu.async_copy(coords_slice(1), cb1, csem1)

        def outer(ob, carry):
            for par in (0, 1):
                b = ob * 2 + par
                X, Y = par, 1 - par

                @pl.when(b + 1 < NB)
                def _():
                    pltpu.make_async_copy(coords_slice(0),
                                          cbuf[Y], csem[Y]).wait()
                    compute_idx(cbuf[Y], idxb[Y], wbuf[Y])
                    issue_gathers(idxb[Y], rowsb[Y], gsem[Y])

                @pl.when(b + 2 < NB)
                def _():
                    pltpu.async_copy(coords_slice(b + 2), cbuf[X], csem[X])

                drain_gathers(rowsb[X], gsem[X])

                @pl.when(b >= 2)
                def _():
                    pltpu.make_async_copy(outbb[X], out_slice(0),
                                          osem[X]).wait()

                compute_sum(wbuf[X], rowsb[X], outbb[X])
                pltpu.async_copy(outbb[X], out_slice(b), osem[X])
            return carry

        lax.fori_loop(0, NB // 2, outer, 0, unroll=False)

        pltpu.make_async_copy(outbb[0], out_slice(0), osem0).wait()
        pltpu.make_async_copy(outbb[1], out_slice(0), osem1).wait()

    return gs_kernel


@functools.lru_cache(maxsize=None)
def _make_transpose_kernel(N, C, D, H, W):
    HW = H * W
    DHW = D * HW
    PK = 128 // C
    RB = HW // PK

    def body(in_ref, out_ref):
        x = in_ref[0, :, 0]
        x2 = x.reshape(C, HW)
        for a in range(HW // 512):
            out_ref[pl.ds(128 * a, 128), :] = jnp.concatenate(
                [x2[:, 512 * a + 128 * b:512 * a + 128 * (b + 1)].T
                 for b in range(PK)], axis=1)

    return pl.pallas_call(
        body,
        grid=(N, D),
        in_specs=[pl.BlockSpec((1, C, 1, H, W), lambda n, d: (n, 0, d, 0, 0))],
        out_specs=pl.BlockSpec((RB, PK * C), lambda n, d: (n * D + d, 0)),
        out_shape=jax.ShapeDtypeStruct((N * DHW // PK, PK * C), jnp.float32),
    )


def kernel(img, grid):
    N, C, D, H, W = img.shape
    N2, Do, Ho, Wo, three = grid.shape
    assert N2 == N and three == 3
    SQ = Do * Ho * Wo
    Q = N * SQ
    BQ = 128
    DHW = D * H * W
    assert (DHW * C) % 128 == 0 and 128 % C == 0
    tab = _make_transpose_kernel(N, C, D, H, W)(img).reshape(N * DHW, C)
    g3 = jnp.moveaxis(grid, -1, 0).reshape(3, Q)
    coords = jnp.moveaxis(g3.reshape(3, Q // BQ, BQ), 0, 1).reshape(
        Q // BQ, 3 * BQ)
    f = _make_gs_kernel(N, C, D, H, W, SQ)
    out = f(tab, coords)
    return out.reshape(N, C, Do, Ho, Wo)
